# baseline (device time: 541285 ns/iter reference)
import jax
import jax.numpy as jnp
from jax import lax
from jax.experimental import pallas as pl
from jax.experimental.pallas import tpu as pltpu

K_LOC = 4096
M = 4096
M_HALF = 2048
N = 8192
N_HALF = 4096

NC = 16
C = N_HALF // NC

_MESH = pl.DeviceIdType.MESH


def _fused_body(x16_ref, dy_blk, out_ref,
                xv, p, rsv, rbuf,
                rs_send, rs_recv, ag_send, ag_recv,
                x_sem, cp_sem, credit):
    mx = lax.axis_index("x")
    my = lax.axis_index("y")
    y_nbr = (mx, 1 - my)
    x_nbr = (1 - mx, my)
    k = pl.program_id(0)

    def rs_rdma(j, slot):
        return pltpu.make_async_remote_copy(
            src_ref=p.at[slot, pl.ds((1 - my) * M_HALF, M_HALF), :],
            dst_ref=rsv.at[slot],
            send_sem=rs_send.at[j], recv_sem=rs_recv.at[j],
            device_id=y_nbr, device_id_type=_MESH)

    def ag_rdma(j, slot2):
        return pltpu.make_async_remote_copy(
            src_ref=rbuf.at[slot2],
            dst_ref=out_ref.at[:, pl.ds(mx * N_HALF + j * C, C)],
            send_sem=ag_send.at[j], recv_sem=ag_recv.at[j],
            device_id=x_nbr, device_id_type=_MESH)

    def out_cp(j, slot2):
        return pltpu.make_async_copy(
            rbuf.at[slot2],
            out_ref.at[:, pl.ds(mx * N_HALF + j * C, C)],
            cp_sem.at[slot2])

    @pl.when(k == 0)
    def _entry():
        barrier = pltpu.get_barrier_semaphore()
        pl.semaphore_signal(barrier, inc=1, device_id=y_nbr,
                            device_id_type=_MESH)
        pl.semaphore_signal(barrier, inc=1, device_id=x_nbr,
                            device_id_type=_MESH)
        pl.semaphore_wait(barrier, 2)
        pltpu.make_async_copy(x16_ref, xv, x_sem).start()
        pltpu.make_async_copy(x16_ref, xv, x_sem).wait()

    @pl.when(k < NC)
    def _compute_and_send():
        slot = k % 3

        @pl.when(k >= 3)
        def _():
            rs_rdma(k - 3, slot).wait_send()
            pl.semaphore_wait(credit, 1)

        for mt in range(2):
            p[slot, mt * M_HALF:(mt + 1) * M_HALF, :] = (
                lax.dot_general(
                    xv[:, mt * M_HALF:(mt + 1) * M_HALF], dy_blk[...],
                    dimension_numbers=(((0,), (0,)), ((), ())),
                    preferred_element_type=jnp.float32))
        rs_rdma(k, slot).start()

    @pl.when((k >= 2) & (k <= NC + 1))
    def _consume():
        j = k - 2
        slot = j % 3
        slot2 = j % 2
        rs_rdma(j, slot).wait_recv()

        @pl.when(j >= 2)
        def _():
            ag_rdma(j - 2, slot2).wait_send()
            out_cp(j - 2, slot2).wait()

        rbuf[slot2] = (p[slot, pl.ds(my * M_HALF, M_HALF), :]
                       + rsv[slot])

        @pl.when(j <= NC - 4)
        def _():
            pl.semaphore_signal(credit, inc=1, device_id=y_nbr,
                                device_id_type=_MESH)

        out_cp(j, slot2).start()
        ag_rdma(j, slot2).start()

    @pl.when(k == NC + 1)
    def _drain():
        rs_rdma(NC - 3, (NC - 3) % 3).wait_send()
        rs_rdma(NC - 2, (NC - 2) % 3).wait_send()
        rs_rdma(NC - 1, (NC - 1) % 3).wait_send()
        ag_rdma(NC - 2, 0).wait_send()
        ag_rdma(NC - 1, 1).wait_send()
        out_cp(NC - 2, 0).wait()
        out_cp(NC - 1, 1).wait()
        for j in range(NC):
            ag_rdma(j, j % 2).wait_recv()


def _fused(x16, dy16):
    return pl.pallas_call(
        _fused_body,
        grid=(NC + 2,),
        out_shape=jax.ShapeDtypeStruct((M_HALF, N), jnp.float32),
        in_specs=[
            pl.BlockSpec(memory_space=pl.ANY),
            pl.BlockSpec((K_LOC, C),
                         lambda k: (0, jnp.minimum(k, NC - 1))),
        ],
        out_specs=pl.BlockSpec(memory_space=pl.ANY),
        scratch_shapes=[
            pltpu.VMEM((K_LOC, M), jnp.bfloat16),
            pltpu.VMEM((3, M, C), jnp.float32),
            pltpu.VMEM((3, M_HALF, C), jnp.float32),
            pltpu.VMEM((2, M_HALF, C), jnp.float32),
            pltpu.SemaphoreType.DMA((NC,)),
            pltpu.SemaphoreType.DMA((NC,)),
            pltpu.SemaphoreType.DMA((NC,)),
            pltpu.SemaphoreType.DMA((NC,)),
            pltpu.SemaphoreType.DMA,
            pltpu.SemaphoreType.DMA((2,)),
            pltpu.SemaphoreType.REGULAR,
        ],
        compiler_params=pltpu.CompilerParams(
            collective_id=0,
            vmem_limit_bytes=63 * 1024 * 1024,
            dimension_semantics=("arbitrary",),
            fuse_transposed_lhs_in_matmul=True),
    )(x16, dy16)


def kernel(x, dy):
    mx = lax.axis_index("x")
    x16 = x.astype(jnp.bfloat16)
    dy16 = lax.dynamic_slice_in_dim(dy, mx * N_HALF, N_HALF,
                                    axis=1).astype(jnp.bfloat16)
    return _fused(x16, dy16)


# device time: 540779 ns/iter; 1.0009x vs baseline; 1.0009x over previous
import jax
import jax.numpy as jnp
from jax import lax
from jax.experimental import pallas as pl
from jax.experimental.pallas import tpu as pltpu

K_LOC = 4096
M = 4096
M_HALF = 2048
N = 8192
N_HALF = 4096

NC = 16
C = N_HALF // NC

_MESH = pl.DeviceIdType.MESH


def _fused_body(x16_ref, dy_blk, out_ref,
                xv, p, rsv, rbuf,
                rs_send, rs_recv, ag_send, ag_recv,
                x_sem, cp_sem, credit):
    mx = lax.axis_index("x")
    my = lax.axis_index("y")
    y_nbr = (mx, 1 - my)
    x_nbr = (1 - mx, my)
    k = pl.program_id(0)

    def rs_rdma(j, slot):
        return pltpu.make_async_remote_copy(
            src_ref=p.at[slot, pl.ds((1 - my) * M_HALF, M_HALF), :],
            dst_ref=rsv.at[slot],
            send_sem=rs_send.at[j], recv_sem=rs_recv.at[j],
            device_id=y_nbr, device_id_type=_MESH)

    def ag_rdma(j, slot2):
        return pltpu.make_async_remote_copy(
            src_ref=rbuf.at[slot2],
            dst_ref=out_ref.at[:, pl.ds(mx * N_HALF + j * C, C)],
            send_sem=ag_send.at[j], recv_sem=ag_recv.at[j],
            device_id=x_nbr, device_id_type=_MESH)

    def out_cp(j, slot2):
        return pltpu.make_async_copy(
            rbuf.at[slot2],
            out_ref.at[:, pl.ds(mx * N_HALF + j * C, C)],
            cp_sem.at[slot2])

    @pl.when(k == 0)
    def _entry():
        barrier = pltpu.get_barrier_semaphore()
        pl.semaphore_signal(barrier, inc=1, device_id=y_nbr,
                            device_id_type=_MESH)
        pl.semaphore_signal(barrier, inc=1, device_id=x_nbr,
                            device_id_type=_MESH)
        pl.semaphore_wait(barrier, 2)
        pltpu.make_async_copy(x16_ref, xv, x_sem).start()
        pltpu.make_async_copy(x16_ref, xv, x_sem).wait()

    @pl.when(k < NC)
    def _compute_and_send():
        slot = k % 3

        @pl.when(k >= 3)
        def _():
            rs_rdma(k - 3, slot).wait_send()
            pl.semaphore_wait(credit, 1)

        for mt in range(2):
            p[slot, mt * M_HALF:(mt + 1) * M_HALF, :] = (
                lax.dot_general(
                    xv[mt * M_HALF:(mt + 1) * M_HALF, :], dy_blk[...],
                    dimension_numbers=(((1,), (0,)), ((), ())),
                    preferred_element_type=jnp.float32))
        rs_rdma(k, slot).start()

    @pl.when((k >= 2) & (k <= NC + 1))
    def _consume():
        j = k - 2
        slot = j % 3
        slot2 = j % 2
        rs_rdma(j, slot).wait_recv()

        @pl.when(j >= 2)
        def _():
            ag_rdma(j - 2, slot2).wait_send()
            out_cp(j - 2, slot2).wait()

        rbuf[slot2] = (p[slot, pl.ds(my * M_HALF, M_HALF), :]
                       + rsv[slot])

        @pl.when(j <= NC - 4)
        def _():
            pl.semaphore_signal(credit, inc=1, device_id=y_nbr,
                                device_id_type=_MESH)

        out_cp(j, slot2).start()
        ag_rdma(j, slot2).start()

    @pl.when(k == NC + 1)
    def _drain():
        rs_rdma(NC - 3, (NC - 3) % 3).wait_send()
        rs_rdma(NC - 2, (NC - 2) % 3).wait_send()
        rs_rdma(NC - 1, (NC - 1) % 3).wait_send()
        ag_rdma(NC - 2, 0).wait_send()
        ag_rdma(NC - 1, 1).wait_send()
        out_cp(NC - 2, 0).wait()
        out_cp(NC - 1, 1).wait()
        for j in range(NC):
            ag_rdma(j, j % 2).wait_recv()


def _fused(x16, dy16):
    return pl.pallas_call(
        _fused_body,
        grid=(NC + 2,),
        out_shape=jax.ShapeDtypeStruct((M_HALF, N), jnp.float32),
        in_specs=[
            pl.BlockSpec(memory_space=pl.ANY),
            pl.BlockSpec((K_LOC, C),
                         lambda k: (0, jnp.minimum(k, NC - 1))),
        ],
        out_specs=pl.BlockSpec(memory_space=pl.ANY),
        scratch_shapes=[
            pltpu.VMEM((M, K_LOC), jnp.bfloat16),
            pltpu.VMEM((3, M, C), jnp.float32),
            pltpu.VMEM((3, M_HALF, C), jnp.float32),
            pltpu.VMEM((2, M_HALF, C), jnp.float32),
            pltpu.SemaphoreType.DMA((NC,)),
            pltpu.SemaphoreType.DMA((NC,)),
            pltpu.SemaphoreType.DMA((NC,)),
            pltpu.SemaphoreType.DMA((NC,)),
            pltpu.SemaphoreType.DMA,
            pltpu.SemaphoreType.DMA((2,)),
            pltpu.SemaphoreType.REGULAR,
        ],
        compiler_params=pltpu.CompilerParams(
            collective_id=0,
            vmem_limit_bytes=63 * 1024 * 1024,
            dimension_semantics=("arbitrary",)),
    )(x16, dy16)


def kernel(x, dy):
    mx = lax.axis_index("x")
    x16 = x.astype(jnp.bfloat16).T
    dy16 = lax.dynamic_slice_in_dim(dy, mx * N_HALF, N_HALF,
                                    axis=1).astype(jnp.bfloat16)
    return _fused(x16, dy16)


# device time: 521308 ns/iter; 1.0383x vs baseline; 1.0374x over previous
import jax
import jax.numpy as jnp
from jax import lax
from jax.experimental import pallas as pl
from jax.experimental.pallas import tpu as pltpu

K_LOC = 4096
M = 4096
M_HALF = 2048
N = 8192
N_HALF = 4096

NC = 16
C = N_HALF // NC

_MESH = pl.DeviceIdType.MESH


def _fused_body(x_ref, dy_blk, out_ref,
                xv, p, rsv, rbuf,
                rs_send, rs_recv, ag_send, ag_recv,
                x_sems, cp_sem, credit):
    mx = lax.axis_index("x")
    my = lax.axis_index("y")
    y_nbr = (mx, 1 - my)
    x_nbr = (1 - mx, my)
    k = pl.program_id(0)

    def rs_rdma(j, slot):
        return pltpu.make_async_remote_copy(
            src_ref=p.at[slot, pl.ds((1 - my) * M_HALF, M_HALF), :],
            dst_ref=rsv.at[slot],
            send_sem=rs_send.at[j], recv_sem=rs_recv.at[j],
            device_id=y_nbr, device_id_type=_MESH)

    def ag_rdma(j, slot2):
        return pltpu.make_async_remote_copy(
            src_ref=rbuf.at[slot2],
            dst_ref=out_ref.at[:, pl.ds(mx * N_HALF + j * C, C)],
            send_sem=ag_send.at[j], recv_sem=ag_recv.at[j],
            device_id=x_nbr, device_id_type=_MESH)

    def out_cp(j, slot2):
        return pltpu.make_async_copy(
            rbuf.at[slot2],
            out_ref.at[:, pl.ds(mx * N_HALF + j * C, C)],
            cp_sem.at[slot2])

    @pl.when(k == 0)
    def _entry():
        barrier = pltpu.get_barrier_semaphore()
        pl.semaphore_signal(barrier, inc=1, device_id=y_nbr,
                            device_id_type=_MESH)
        pl.semaphore_signal(barrier, inc=1, device_id=x_nbr,
                            device_id_type=_MESH)
        pl.semaphore_wait(barrier, 2)
        def x_strip(j):
            cp = pltpu.make_async_copy(
                x_ref.at[:, pl.ds(j * C, C)], p.at[j % 3],
                x_sems.at[j % 3])
            cp.start()
            return cp

        strips = [x_strip(0), x_strip(1), x_strip(2)]
        for j in range(16):
            strips[j % 3].wait()
            xv[:, j * C:(j + 1) * C] = p[j % 3].astype(jnp.bfloat16)
            if j + 3 < 16:
                strips[j % 3] = x_strip(j + 3)

    @pl.when(k < NC)
    def _compute_and_send():
        slot = k % 3

        @pl.when(k >= 3)
        def _():
            rs_rdma(k - 3, slot).wait_send()
            pl.semaphore_wait(credit, 1)

        for mt in range(2):
            p[slot, mt * M_HALF:(mt + 1) * M_HALF, :] = (
                lax.dot_general(
                    xv[:, mt * M_HALF:(mt + 1) * M_HALF], dy_blk[...],
                    dimension_numbers=(((0,), (0,)), ((), ())),
                    preferred_element_type=jnp.float32))
        rs_rdma(k, slot).start()

    @pl.when((k >= 2) & (k <= NC + 1))
    def _consume():
        j = k - 2
        slot = j % 3
        slot2 = j % 2
        rs_rdma(j, slot).wait_recv()

        @pl.when(j >= 2)
        def _():
            ag_rdma(j - 2, slot2).wait_send()
            out_cp(j - 2, slot2).wait()

        rbuf[slot2] = (p[slot, pl.ds(my * M_HALF, M_HALF), :]
                       + rsv[slot])

        @pl.when(j <= NC - 4)
        def _():
            pl.semaphore_signal(credit, inc=1, device_id=y_nbr,
                                device_id_type=_MESH)

        out_cp(j, slot2).start()
        ag_rdma(j, slot2).start()

    @pl.when(k == NC + 1)
    def _drain():
        rs_rdma(NC - 3, (NC - 3) % 3).wait_send()
        rs_rdma(NC - 2, (NC - 2) % 3).wait_send()
        rs_rdma(NC - 1, (NC - 1) % 3).wait_send()
        ag_rdma(NC - 2, 0).wait_send()
        ag_rdma(NC - 1, 1).wait_send()
        out_cp(NC - 2, 0).wait()
        out_cp(NC - 1, 1).wait()
        for j in range(NC):
            ag_rdma(j, j % 2).wait_recv()


def _fused(x16, dy16):
    return pl.pallas_call(
        _fused_body,
        grid=(NC + 2,),
        out_shape=jax.ShapeDtypeStruct((M_HALF, N), jnp.float32),
        in_specs=[
            pl.BlockSpec(memory_space=pl.ANY),
            pl.BlockSpec((K_LOC, C),
                         lambda k: (0, jnp.minimum(k, NC - 1))),
        ],
        out_specs=pl.BlockSpec(memory_space=pl.ANY),
        scratch_shapes=[
            pltpu.VMEM((K_LOC, M), jnp.bfloat16),
            pltpu.VMEM((3, M, C), jnp.float32),
            pltpu.VMEM((3, M_HALF, C), jnp.float32),
            pltpu.VMEM((2, M_HALF, C), jnp.float32),
            pltpu.SemaphoreType.DMA((NC,)),
            pltpu.SemaphoreType.DMA((NC,)),
            pltpu.SemaphoreType.DMA((NC,)),
            pltpu.SemaphoreType.DMA((NC,)),
            pltpu.SemaphoreType.DMA((3,)),
            pltpu.SemaphoreType.DMA((2,)),
            pltpu.SemaphoreType.REGULAR,
        ],
        compiler_params=pltpu.CompilerParams(
            collective_id=0,
            vmem_limit_bytes=63 * 1024 * 1024,
            dimension_semantics=("arbitrary",),
            fuse_transposed_lhs_in_matmul=True),
    )(x16, dy16)


def kernel(x, dy):
    mx = lax.axis_index("x")
    dy16 = lax.dynamic_slice_in_dim(dy, mx * N_HALF, N_HALF,
                                    axis=1).astype(jnp.bfloat16)
    return _fused(x, dy16)
